# baseline (speedup 1.0000x reference)
import functools

import jax
import jax.numpy as jnp
import numpy as np
from jax import lax
from jax.experimental import pallas as pl
from jax.experimental.pallas import tpu as pltpu
from jax.experimental.pallas import tpu_sc as plsc

N_NODES = 10000
NPAD = 10240
E_EDGES = 320000
H_DIM = 128
OUT_D = 64
GENES_X_DIM = 64000

NW = 32
EPW = E_EDGES // NW
CHUNK = 80
SUP = 1
SUPE = SUP * CHUNK
NSUP = EPW // SUPE
RPW = EPW // CHUNK
RPT = NPAD // 16

_PERM = np.concatenate(
    [np.concatenate([32 * g + 2 * np.arange(16),
                     32 * g + 2 * np.arange(16) + 1]) for g in range(4)])


def _logits_tail(h, asw_ref, adw_ref, as_ref, ad_ref, m_ref, mx_ref, step, last):
    as_blk = jnp.dot(h, asw_ref[...], preferred_element_type=jnp.float32)
    ad_blk = jnp.dot(h, adw_ref[...], preferred_element_type=jnp.float32)
    as_ref[...] = as_blk
    ad_ref[...] = ad_blk
    bmax_s = jnp.max(as_blk)
    bmax_d = jnp.max(ad_blk)

    @pl.when(step == 0)
    def _():
        mx_ref[0] = bmax_s
        mx_ref[1] = bmax_d

    mx_ref[0] = jnp.maximum(mx_ref[0], bmax_s)
    mx_ref[1] = jnp.maximum(mx_ref[1], bmax_d)

    @pl.when(step == last)
    def _():
        m_ref[...] = jnp.full((8, 128), jnp.maximum(mx_ref[0] + mx_ref[1], 0.0))


def _proj_body(x_ref, w_ref, asw_ref, adw_ref, h_ref, as_ref, ad_ref, m_ref,
               mx_ref):
    h = jnp.dot(x_ref[...], w_ref[...], preferred_element_type=jnp.float32)
    h_ref[...] = h.astype(jnp.bfloat16)
    i = pl.program_id(0)
    _logits_tail(h, asw_ref, adw_ref, as_ref, ad_ref, m_ref, mx_ref,
                 i, pl.num_programs(0) - 1)


def _project(x, W, a_s, a_d):
    n = x.shape[0]
    blk = 1000 if n == N_NODES else 1024
    return pl.pallas_call(
        _proj_body,
        grid=(n // blk,),
        in_specs=[
            pl.BlockSpec((blk, H_DIM), lambda i: (i, 0)),
            pl.BlockSpec((H_DIM, H_DIM), lambda i: (0, 0)),
            pl.BlockSpec((H_DIM, 1), lambda i: (0, 0)),
            pl.BlockSpec((H_DIM, 1), lambda i: (0, 0)),
        ],
        out_specs=[
            pl.BlockSpec((blk, H_DIM), lambda i: (i, 0)),
            pl.BlockSpec((blk, 1), lambda i: (i, 0)),
            pl.BlockSpec((blk, 1), lambda i: (i, 0)),
            pl.BlockSpec((8, 128), lambda i: (0, 0)),
        ],
        out_shape=[
            jax.ShapeDtypeStruct((n, H_DIM), jnp.bfloat16),
            jax.ShapeDtypeStruct((n, 1), jnp.float32),
            jax.ShapeDtypeStruct((n, 1), jnp.float32),
            jax.ShapeDtypeStruct((8, 128), jnp.float32),
        ],
        scratch_shapes=[pltpu.SMEM((2,), jnp.float32)],
    )(x, W, a_s.reshape(H_DIM, 1), a_d.reshape(H_DIM, 1))


def _fin_proj_body(num_ref, den_ref, b_ref, w_ref, asw_ref, adw_ref,
                   h_ref, as_ref, ad_ref, m_ref, mx_ref):
    num = num_ref[0] + num_ref[1]
    den = den_ref[0] + den_ref[1] + 1e-16
    hprev = jnp.maximum(num / den + b_ref[...], 0.0)
    h = jnp.dot(hprev, w_ref[...], preferred_element_type=jnp.float32)
    h_ref[...] = h.astype(jnp.bfloat16)
    i = pl.program_id(0)
    _logits_tail(h, asw_ref, adw_ref, as_ref, ad_ref, m_ref, mx_ref,
                 i, pl.num_programs(0) - 1)


def _finalize_project(numer, denom, b, W, a_s, a_d):
    blk = 1024
    return pl.pallas_call(
        _fin_proj_body,
        grid=(NPAD // blk,),
        in_specs=[
            pl.BlockSpec((2, blk, H_DIM), lambda i: (0, i, 0)),
            pl.BlockSpec((2, blk, 1), lambda i: (0, i, 0)),
            pl.BlockSpec((1, H_DIM), lambda i: (0, 0)),
            pl.BlockSpec((H_DIM, H_DIM), lambda i: (0, 0)),
            pl.BlockSpec((H_DIM, 1), lambda i: (0, 0)),
            pl.BlockSpec((H_DIM, 1), lambda i: (0, 0)),
        ],
        out_specs=[
            pl.BlockSpec((blk, H_DIM), lambda i: (i, 0)),
            pl.BlockSpec((blk, 1), lambda i: (i, 0)),
            pl.BlockSpec((blk, 1), lambda i: (i, 0)),
            pl.BlockSpec((8, 128), lambda i: (0, 0)),
        ],
        out_shape=[
            jax.ShapeDtypeStruct((NPAD, H_DIM), jnp.bfloat16),
            jax.ShapeDtypeStruct((NPAD, 1), jnp.float32),
            jax.ShapeDtypeStruct((NPAD, 1), jnp.float32),
            jax.ShapeDtypeStruct((8, 128), jnp.float32),
        ],
        scratch_shapes=[pltpu.SMEM((2,), jnp.float32)],
    )(numer, denom.reshape(2, NPAD, 1), b.reshape(1, H_DIM),
      W, a_s.reshape(H_DIM, 1), a_d.reshape(H_DIM, 1))


def _fin_lin_body(num_ref, den_ref, b_ref, w_ref, bl_ref, o_ref):
    num = num_ref[0] + num_ref[1]
    den = den_ref[0] + den_ref[1] + 1e-16
    hprev = jnp.maximum(num / den + b_ref[...], 0.0)
    o_ref[...] = jnp.dot(hprev, w_ref[...],
                         preferred_element_type=jnp.float32) + bl_ref[...]


def _finalize_linear(numer, denom, b, Wl, bl):
    blk = 2000
    return pl.pallas_call(
        _fin_lin_body,
        grid=(N_NODES // blk,),
        in_specs=[
            pl.BlockSpec((2, blk, H_DIM), lambda i: (0, i, 0)),
            pl.BlockSpec((2, blk, 1), lambda i: (0, i, 0)),
            pl.BlockSpec((1, H_DIM), lambda i: (0, 0)),
            pl.BlockSpec((H_DIM, OUT_D), lambda i: (0, 0)),
            pl.BlockSpec((1, OUT_D), lambda i: (0, 0)),
        ],
        out_specs=pl.BlockSpec((blk, OUT_D), lambda i: (i, 0)),
        out_shape=jax.ShapeDtypeStruct((N_NODES, OUT_D), jnp.float32),
    )(numer, denom.reshape(2, NPAD, 1), b.reshape(1, H_DIM),
      Wl, bl.reshape(1, OUT_D))


def _make_sc_edge(table_size):
    mesh = plsc.VectorSubcoreMesh(core_axis_name="c", subcore_axis_name="s")

    @functools.partial(
        pl.kernel,
        out_type=[
            jax.ShapeDtypeStruct((2, NPAD, H_DIM), jnp.float32),
            jax.ShapeDtypeStruct((2, NPAD), jnp.float32),
        ],
        mesh=mesh,
        compiler_params=pltpu.CompilerParams(needs_layout_passes=False,
                                             use_tc_tiling_on_sc=False),
        scratch_types=(
            [
                pltpu.VMEM((NPAD,), jnp.float32),
                pltpu.VMEM((NPAD,), jnp.float32),
                pltpu.VMEM((16,), jnp.float32),
                pltpu.VMEM((2, CHUNK, H_DIM // 2), jnp.int32),
                pltpu.VMEM((CHUNK, H_DIM), jnp.float32),
            ]
            + [pltpu.VMEM((CHUNK,), jnp.int32) for _ in range(4)]
            + [pltpu.VMEM((CHUNK,), jnp.int32) for _ in range(4)]
            + [pltpu.VMEM((CHUNK,), jnp.float32) for _ in range(4)]
            + [
                pltpu.VMEM_SHARED((NPAD, H_DIM), jnp.float32),
                pltpu.VMEM_SHARED((NPAD,), jnp.float32),
                pltpu.SemaphoreType.DMA,
                pltpu.SemaphoreType.DMA,
                pltpu.SemaphoreType.DMA,
                pltpu.SemaphoreType.DMA,
                pltpu.SemaphoreType.DMA,
                pltpu.SemaphoreType.DMA,
            ]
        ),
    )
    def sc_edge(h_hbm, as_hbm, ad_hbm, m_hbm, src_hbm, dst_hbm,
                numer_out, denom_out,
                as_v, ad_v, m_v, rows_v, stage_v, *rest):
        srcs = list(rest[0:4])
        dsts = list(rest[4:8])
        ws = list(rest[8:12])
        numer_sh, denom_sh, sem_g0, sem_g1, si0, si1, si2, si3 = rest[12:]
        sems = [sem_g0, sem_g1]
        sem_i = [si0, si1, si2, si3]
        cid = lax.axis_index("c")
        sid = lax.axis_index("s")
        zeros16 = jnp.zeros((16,), jnp.float32)

        pltpu.sync_copy(as_hbm, as_v.at[pl.ds(0, table_size)])
        pltpu.sync_copy(ad_hbm, ad_v.at[pl.ds(0, table_size)])
        pltpu.sync_copy(m_hbm.at[0, pl.ds(0, 16)], m_v)
        m_val = m_v[...]

        def zbody(i, _):
            stage_v[i // 8, pl.ds((i % 8) * 16, 16)] = zeros16
            return 0
        lax.fori_loop(0, CHUNK * 8, zbody, 0)
        for k in range(CHUNK // 16):
            ws[0][pl.ds(k * 16, 16)] = zeros16
        for t in range(RPT // CHUNK):
            pltpu.sync_copy(stage_v,
                            numer_sh.at[pl.ds(sid * RPT + t * CHUNK, CHUNK)])
            pltpu.sync_copy(ws[0],
                            denom_sh.at[pl.ds(sid * RPT + t * CHUNK, CHUNK)])
        plsc.subcore_barrier()

        base = (cid * 16 + sid) * EPW

        def load_idx(c, m4, sync):
            eb = base + c * CHUNK
            if sync:
                pltpu.sync_copy(src_hbm.at[pl.ds(eb, CHUNK)], srcs[m4])
                pltpu.sync_copy(dst_hbm.at[pl.ds(eb, CHUNK)], dsts[m4])
            else:
                pltpu.async_copy(src_hbm.at[pl.ds(eb, CHUNK)], srcs[m4],
                                 sem_i[m4])
                pltpu.async_copy(dst_hbm.at[pl.ds(eb, CHUNK)], dsts[m4],
                                 sem_i[m4])

        def drain_idx(m4):
            pltpu.make_async_copy(src_hbm.at[pl.ds(0, CHUNK)],
                                  srcs[m4], sem_i[m4]).wait()
            pltpu.make_async_copy(dst_hbm.at[pl.ds(0, CHUNK)],
                                  dsts[m4], sem_i[m4]).wait()

        def fire(m4, m2):
            pltpu.async_copy(h_hbm.at[srcs[m4]], rows_v.at[m2], sems[m2])
            for k in range(CHUNK // 16):
                s16 = srcs[m4][pl.ds(k * 16, 16)]
                d16 = dsts[m4][pl.ds(k * 16, 16)]
                z = (plsc.load_gather(as_v, [s16])
                     + plsc.load_gather(ad_v, [d16]))
                e = jnp.maximum(z, 0.2 * z)
                ws[m4][pl.ds(k * 16, 16)] = jnp.exp(e - m_val)

        def process(m4, m2):
            pltpu.make_async_copy(h_hbm.at[srcs[m4]],
                                  rows_v.at[m2], sems[m2]).wait()

            def mulbody(t, _):
                for u in range(4):
                    ei = 4 * t + u
                    wspl = plsc.load_gather(
                        ws[m4], [jnp.full((16,), ei, jnp.int32)])
                    for g in range(H_DIM // 32):
                        vi = rows_v[m2, ei, pl.ds(g * 16, 16)]
                        v32 = plsc.bitcast(vi, jnp.bfloat16)
                        a, b = plsc.unpack(v32, format=plsc.PackFormat.INTERLEAVED)
                        stage_v[ei, pl.ds(g * 32, 16)] = a * wspl
                        stage_v[ei, pl.ds(g * 32 + 16, 16)] = b * wspl
                return 0
            lax.fori_loop(0, CHUNK // 4, mulbody, 0)

            pltpu.sync_copy(stage_v, numer_sh.at[dsts[m4]], add=True)
            pltpu.sync_copy(ws[m4], denom_sh.at[dsts[m4]], add=True)

        load_idx(0, 0, sync=True)
        load_idx(1, 1, sync=True)
        load_idx(2, 2, sync=False)
        load_idx(3, 3, sync=False)
        fire(0, 0)
        fire(1, 1)

        def quad_body(k, _):
            c = 4 * k
            for j in range(4):
                process(j, j % 2)
                load_idx(c + j + 4, j, sync=False)
                drain_idx((j + 2) % 4)
                fire((j + 2) % 4, j % 2)
            return 0
        lax.fori_loop(0, RPW // 4 - 1, quad_body, 0)

        process(0, 0)
        load_idx(RPW - 1, 0, sync=False)
        drain_idx(2)
        fire(2, 0)
        process(1, 1)
        drain_idx(3)
        fire(3, 1)
        process(2, 0)
        drain_idx(0)
        fire(0, 0)
        process(3, 1)
        process(0, 0)

        plsc.subcore_barrier()
        r0 = sid * RPT
        pltpu.sync_copy(numer_sh.at[pl.ds(r0, RPT)],
                        numer_out.at[cid, pl.ds(r0, RPT)])
        pltpu.sync_copy(denom_sh.at[pl.ds(r0, RPT)],
                        denom_out.at[cid, pl.ds(r0, RPT)])

    return sc_edge


_sc_edge_l1 = _make_sc_edge(N_NODES)
_sc_edge_l2 = _make_sc_edge(NPAD)


@jax.jit
def kernel(x, edge_index, edge_attr, batch, W1, a_s1, a_d1, b1,
           W2, a_s2, a_d2, b2, Wl, bl):
    src = edge_index[0]
    dst = edge_index[1]

    h1, as1, ad1, m1 = _project(x, W1, a_s1, a_d1)
    h1i = lax.bitcast_convert_type(
        h1.reshape(N_NODES, H_DIM // 2, 2), jnp.int32)
    numer1, denom1 = _sc_edge_l1(h1i, as1.reshape(-1), ad1.reshape(-1), m1,
                                 src, dst)
    h2, as2, ad2, m2 = _finalize_project(numer1, denom1, b1[_PERM],
                                         W2[_PERM], a_s2, a_d2)
    h2i = lax.bitcast_convert_type(h2.reshape(NPAD, H_DIM // 2, 2), jnp.int32)
    numer2, denom2 = _sc_edge_l2(h2i, as2.reshape(-1), ad2.reshape(-1), m2,
                                 src, dst)
    out = _finalize_linear(numer2, denom2, b2[_PERM], Wl[_PERM], bl)
    return out.reshape(-1).reshape(-1, GENES_X_DIM)


# speedup vs baseline: 1.7237x; 1.7237x over previous
import functools

import jax
import jax.numpy as jnp
from jax import lax
from jax.experimental import pallas as pl
from jax.experimental.pallas import tpu as pltpu
from jax.experimental.pallas import tpu_sc as plsc

N_NODES = 10000
NPAD = 10240
E_EDGES = 320000
H_DIM = 128
OUT_D = 64
GENES_X_DIM = 64000

NW = 32
EPW = E_EDGES // NW
CHUNK = 80
SUP = 1
SUPE = SUP * CHUNK
NSUP = EPW // SUPE
RPW = EPW // CHUNK
RPT = NPAD // 16


def _logits_tail(h, asw_ref, adw_ref, as_ref, ad_ref, m_ref, mx_ref, step, last):
    as_blk = jnp.dot(h, asw_ref[...], preferred_element_type=jnp.float32)
    ad_blk = jnp.dot(h, adw_ref[...], preferred_element_type=jnp.float32)
    as_ref[...] = as_blk
    ad_ref[...] = ad_blk
    bmax_s = jnp.max(as_blk)
    bmax_d = jnp.max(ad_blk)

    @pl.when(step == 0)
    def _():
        mx_ref[0] = bmax_s
        mx_ref[1] = bmax_d

    mx_ref[0] = jnp.maximum(mx_ref[0], bmax_s)
    mx_ref[1] = jnp.maximum(mx_ref[1], bmax_d)

    @pl.when(step == last)
    def _():
        m_ref[...] = jnp.full((8, 128), jnp.maximum(mx_ref[0] + mx_ref[1], 0.0))


def _proj_body(x_ref, w_ref, asw_ref, adw_ref, h_ref, as_ref, ad_ref, m_ref,
               mx_ref):
    h = jnp.dot(x_ref[...], w_ref[...], preferred_element_type=jnp.float32)
    h_ref[...] = h
    i = pl.program_id(0)
    _logits_tail(h, asw_ref, adw_ref, as_ref, ad_ref, m_ref, mx_ref,
                 i, pl.num_programs(0) - 1)


def _project(x, W, a_s, a_d):
    n = x.shape[0]
    blk = 1000 if n == N_NODES else 1024
    return pl.pallas_call(
        _proj_body,
        grid=(n // blk,),
        in_specs=[
            pl.BlockSpec((blk, H_DIM), lambda i: (i, 0)),
            pl.BlockSpec((H_DIM, H_DIM), lambda i: (0, 0)),
            pl.BlockSpec((H_DIM, 1), lambda i: (0, 0)),
            pl.BlockSpec((H_DIM, 1), lambda i: (0, 0)),
        ],
        out_specs=[
            pl.BlockSpec((blk, H_DIM), lambda i: (i, 0)),
            pl.BlockSpec((blk, 1), lambda i: (i, 0)),
            pl.BlockSpec((blk, 1), lambda i: (i, 0)),
            pl.BlockSpec((8, 128), lambda i: (0, 0)),
        ],
        out_shape=[
            jax.ShapeDtypeStruct((n, H_DIM), jnp.float32),
            jax.ShapeDtypeStruct((n, 1), jnp.float32),
            jax.ShapeDtypeStruct((n, 1), jnp.float32),
            jax.ShapeDtypeStruct((8, 128), jnp.float32),
        ],
        scratch_shapes=[pltpu.SMEM((2,), jnp.float32)],
    )(x, W, a_s.reshape(H_DIM, 1), a_d.reshape(H_DIM, 1))


def _fin_proj_body(num_ref, den_ref, b_ref, w_ref, asw_ref, adw_ref,
                   h_ref, as_ref, ad_ref, m_ref, mx_ref):
    num = num_ref[0] + num_ref[1]
    den = den_ref[0] + den_ref[1] + 1e-16
    hprev = jnp.maximum(num / den + b_ref[...], 0.0)
    h = jnp.dot(hprev, w_ref[...], preferred_element_type=jnp.float32)
    h_ref[...] = h
    i = pl.program_id(0)
    _logits_tail(h, asw_ref, adw_ref, as_ref, ad_ref, m_ref, mx_ref,
                 i, pl.num_programs(0) - 1)


def _finalize_project(numer, denom, b, W, a_s, a_d):
    blk = 1024
    return pl.pallas_call(
        _fin_proj_body,
        grid=(NPAD // blk,),
        in_specs=[
            pl.BlockSpec((2, blk, H_DIM), lambda i: (0, i, 0)),
            pl.BlockSpec((2, blk, 1), lambda i: (0, i, 0)),
            pl.BlockSpec((1, H_DIM), lambda i: (0, 0)),
            pl.BlockSpec((H_DIM, H_DIM), lambda i: (0, 0)),
            pl.BlockSpec((H_DIM, 1), lambda i: (0, 0)),
            pl.BlockSpec((H_DIM, 1), lambda i: (0, 0)),
        ],
        out_specs=[
            pl.BlockSpec((blk, H_DIM), lambda i: (i, 0)),
            pl.BlockSpec((blk, 1), lambda i: (i, 0)),
            pl.BlockSpec((blk, 1), lambda i: (i, 0)),
            pl.BlockSpec((8, 128), lambda i: (0, 0)),
        ],
        out_shape=[
            jax.ShapeDtypeStruct((NPAD, H_DIM), jnp.float32),
            jax.ShapeDtypeStruct((NPAD, 1), jnp.float32),
            jax.ShapeDtypeStruct((NPAD, 1), jnp.float32),
            jax.ShapeDtypeStruct((8, 128), jnp.float32),
        ],
        scratch_shapes=[pltpu.SMEM((2,), jnp.float32)],
    )(numer, denom.reshape(2, NPAD, 1), b.reshape(1, H_DIM),
      W, a_s.reshape(H_DIM, 1), a_d.reshape(H_DIM, 1))


def _fin_lin_body(num_ref, den_ref, b_ref, w_ref, bl_ref, o_ref):
    num = num_ref[0] + num_ref[1]
    den = den_ref[0] + den_ref[1] + 1e-16
    hprev = jnp.maximum(num / den + b_ref[...], 0.0)
    o_ref[...] = jnp.dot(hprev, w_ref[...],
                         preferred_element_type=jnp.float32) + bl_ref[...]


def _finalize_linear(numer, denom, b, Wl, bl):
    blk = 2000
    return pl.pallas_call(
        _fin_lin_body,
        grid=(N_NODES // blk,),
        in_specs=[
            pl.BlockSpec((2, blk, H_DIM), lambda i: (0, i, 0)),
            pl.BlockSpec((2, blk, 1), lambda i: (0, i, 0)),
            pl.BlockSpec((1, H_DIM), lambda i: (0, 0)),
            pl.BlockSpec((H_DIM, OUT_D), lambda i: (0, 0)),
            pl.BlockSpec((1, OUT_D), lambda i: (0, 0)),
        ],
        out_specs=pl.BlockSpec((blk, OUT_D), lambda i: (i, 0)),
        out_shape=jax.ShapeDtypeStruct((N_NODES, OUT_D), jnp.float32),
    )(numer, denom.reshape(2, NPAD, 1), b.reshape(1, H_DIM),
      Wl, bl.reshape(1, OUT_D))


def _make_sc_edge(table_size):
    mesh = plsc.VectorSubcoreMesh(core_axis_name="c", subcore_axis_name="s")

    @functools.partial(
        pl.kernel,
        out_type=[
            jax.ShapeDtypeStruct((2, NPAD, H_DIM), jnp.float32),
            jax.ShapeDtypeStruct((2, NPAD), jnp.float32),
        ],
        mesh=mesh,
        compiler_params=pltpu.CompilerParams(needs_layout_passes=False),
        scratch_types=(
            [
                pltpu.VMEM((NPAD,), jnp.float32),
                pltpu.VMEM((NPAD,), jnp.float32),
                pltpu.VMEM((16,), jnp.float32),
                pltpu.VMEM((2, CHUNK, H_DIM), jnp.float32),
            ]
            + [pltpu.VMEM((CHUNK,), jnp.int32) for _ in range(4)]
            + [pltpu.VMEM((CHUNK,), jnp.int32) for _ in range(4)]
            + [pltpu.VMEM((CHUNK,), jnp.float32) for _ in range(4)]
            + [
                pltpu.VMEM_SHARED((NPAD, H_DIM), jnp.float32),
                pltpu.VMEM_SHARED((NPAD,), jnp.float32),
                pltpu.SemaphoreType.DMA,
                pltpu.SemaphoreType.DMA,
                pltpu.SemaphoreType.DMA,
                pltpu.SemaphoreType.DMA,
                pltpu.SemaphoreType.DMA,
                pltpu.SemaphoreType.DMA,
            ]
        ),
    )
    def sc_edge(h_hbm, as_hbm, ad_hbm, m_hbm, src_hbm, dst_hbm,
                numer_out, denom_out,
                as_v, ad_v, m_v, rows_v, *rest):
        srcs = list(rest[0:4])
        dsts = list(rest[4:8])
        ws = list(rest[8:12])
        numer_sh, denom_sh, sem_g0, sem_g1, si0, si1, si2, si3 = rest[12:]
        sems = [sem_g0, sem_g1]
        sem_i = [si0, si1, si2, si3]
        cid = lax.axis_index("c")
        sid = lax.axis_index("s")
        zeros16 = jnp.zeros((16,), jnp.float32)

        pltpu.sync_copy(as_hbm, as_v.at[pl.ds(0, table_size)])
        pltpu.sync_copy(ad_hbm, ad_v.at[pl.ds(0, table_size)])
        pltpu.sync_copy(m_hbm.at[0, pl.ds(0, 16)], m_v)
        m_val = m_v[...]

        def zbody(i, _):
            rows_v[0, i // 8, pl.ds((i % 8) * 16, 16)] = zeros16
            return 0
        lax.fori_loop(0, CHUNK * 8, zbody, 0)
        for k in range(CHUNK // 16):
            ws[0][pl.ds(k * 16, 16)] = zeros16
        for t in range(RPT // CHUNK):
            pltpu.sync_copy(rows_v.at[0],
                            numer_sh.at[pl.ds(sid * RPT + t * CHUNK, CHUNK)])
            pltpu.sync_copy(ws[0],
                            denom_sh.at[pl.ds(sid * RPT + t * CHUNK, CHUNK)])
        plsc.subcore_barrier()

        base = (cid * 16 + sid) * EPW

        def load_idx(c, m4, sync):
            eb = base + c * CHUNK
            if sync:
                pltpu.sync_copy(src_hbm.at[pl.ds(eb, CHUNK)], srcs[m4])
                pltpu.sync_copy(dst_hbm.at[pl.ds(eb, CHUNK)], dsts[m4])
            else:
                pltpu.async_copy(src_hbm.at[pl.ds(eb, CHUNK)], srcs[m4],
                                 sem_i[m4])
                pltpu.async_copy(dst_hbm.at[pl.ds(eb, CHUNK)], dsts[m4],
                                 sem_i[m4])

        def drain_idx(m4):
            pltpu.make_async_copy(src_hbm.at[pl.ds(0, CHUNK)],
                                  srcs[m4], sem_i[m4]).wait()
            pltpu.make_async_copy(dst_hbm.at[pl.ds(0, CHUNK)],
                                  dsts[m4], sem_i[m4]).wait()

        def fire(m4, m2):
            pltpu.async_copy(h_hbm.at[srcs[m4]], rows_v.at[m2], sems[m2])
            for k in range(CHUNK // 16):
                s16 = srcs[m4][pl.ds(k * 16, 16)]
                d16 = dsts[m4][pl.ds(k * 16, 16)]
                z = (plsc.load_gather(as_v, [s16])
                     + plsc.load_gather(ad_v, [d16]))
                e = jnp.maximum(z, 0.2 * z)
                ws[m4][pl.ds(k * 16, 16)] = jnp.exp(e - m_val)

        def process(m4, m2):
            pltpu.make_async_copy(h_hbm.at[srcs[m4]],
                                  rows_v.at[m2], sems[m2]).wait()

            def mulbody(t, _):
                for u in range(4):
                    ei = 4 * t + u
                    wspl = plsc.load_gather(
                        ws[m4], [jnp.full((16,), ei, jnp.int32)])
                    for k in range(H_DIM // 16):
                        rows_v[m2, ei, pl.ds(k * 16, 16)] = (
                            rows_v[m2, ei, pl.ds(k * 16, 16)] * wspl)
                return 0
            lax.fori_loop(0, CHUNK // 4, mulbody, 0)

            pltpu.sync_copy(rows_v.at[m2], numer_sh.at[dsts[m4]], add=True)
            pltpu.sync_copy(ws[m4], denom_sh.at[dsts[m4]], add=True)

        load_idx(0, 0, sync=True)
        load_idx(1, 1, sync=True)
        load_idx(2, 2, sync=False)
        load_idx(3, 3, sync=False)
        fire(0, 0)
        fire(1, 1)

        def quad_body(k, _):
            c = 4 * k
            for j in range(4):
                process(j, j % 2)
                load_idx(c + j + 4, j, sync=False)
                drain_idx((j + 2) % 4)
                fire((j + 2) % 4, j % 2)
            return 0
        lax.fori_loop(0, RPW // 4 - 1, quad_body, 0)

        process(0, 0)
        load_idx(RPW - 1, 0, sync=False)
        drain_idx(2)
        fire(2, 0)
        process(1, 1)
        drain_idx(3)
        fire(3, 1)
        process(2, 0)
        drain_idx(0)
        fire(0, 0)
        process(3, 1)
        process(0, 0)

        plsc.subcore_barrier()
        r0 = sid * RPT
        pltpu.sync_copy(numer_sh.at[pl.ds(r0, RPT)],
                        numer_out.at[cid, pl.ds(r0, RPT)])
        pltpu.sync_copy(denom_sh.at[pl.ds(r0, RPT)],
                        denom_out.at[cid, pl.ds(r0, RPT)])

    return sc_edge


_sc_edge_l1 = _make_sc_edge(N_NODES)
_sc_edge_l2 = _make_sc_edge(NPAD)


@jax.jit
def kernel(x, edge_index, edge_attr, batch, W1, a_s1, a_d1, b1,
           W2, a_s2, a_d2, b2, Wl, bl):
    src = edge_index[0]
    dst = edge_index[1]

    h1, as1, ad1, m1 = _project(x, W1, a_s1, a_d1)
    numer1, denom1 = _sc_edge_l1(h1, as1.reshape(-1), ad1.reshape(-1), m1,
                                 src, dst)
    h2, as2, ad2, m2 = _finalize_project(numer1, denom1, b1, W2, a_s2, a_d2)
    numer2, denom2 = _sc_edge_l2(h2, as2.reshape(-1), ad2.reshape(-1), m2,
                                 src, dst)
    out = _finalize_linear(numer2, denom2, b2, Wl, bl)
    return out.reshape(-1).reshape(-1, GENES_X_DIM)


# speedup vs baseline: 1.7972x; 1.0426x over previous
import functools

import jax
import jax.numpy as jnp
from jax import lax
from jax.experimental import pallas as pl
from jax.experimental.pallas import tpu as pltpu
from jax.experimental.pallas import tpu_sc as plsc

N_NODES = 10000
NPAD = 10240
E_EDGES = 320000
H_DIM = 128
OUT_D = 64
GENES_X_DIM = 64000

NW = 32
EPW = E_EDGES // NW
CHUNK = 80
SUP = 1
SUPE = SUP * CHUNK
NSUP = EPW // SUPE
RPW = EPW // CHUNK
RPT = NPAD // 16


def _logits_tail(h, asw_ref, adw_ref, as_ref, ad_ref, m_ref, mx_ref, step, last):
    as_blk = jnp.dot(h, asw_ref[...], preferred_element_type=jnp.float32)
    ad_blk = jnp.dot(h, adw_ref[...], preferred_element_type=jnp.float32)
    as_ref[...] = as_blk
    ad_ref[...] = ad_blk
    bmax_s = jnp.max(as_blk)
    bmax_d = jnp.max(ad_blk)

    @pl.when(step == 0)
    def _():
        mx_ref[0] = bmax_s
        mx_ref[1] = bmax_d

    mx_ref[0] = jnp.maximum(mx_ref[0], bmax_s)
    mx_ref[1] = jnp.maximum(mx_ref[1], bmax_d)

    @pl.when(step == last)
    def _():
        m_ref[...] = jnp.full((8, 128), jnp.maximum(mx_ref[0] + mx_ref[1], 0.0))


def _proj_body(x_ref, w_ref, asw_ref, adw_ref, h_ref, as_ref, ad_ref, m_ref,
               mx_ref):
    h = jnp.dot(x_ref[...], w_ref[...], preferred_element_type=jnp.float32)
    h_ref[...] = h
    i = pl.program_id(0)
    _logits_tail(h, asw_ref, adw_ref, as_ref, ad_ref, m_ref, mx_ref,
                 i, pl.num_programs(0) - 1)


def _project(x, W, a_s, a_d):
    n = x.shape[0]
    blk = 1000 if n == N_NODES else 1024
    return pl.pallas_call(
        _proj_body,
        grid=(n // blk,),
        in_specs=[
            pl.BlockSpec((blk, H_DIM), lambda i: (i, 0)),
            pl.BlockSpec((H_DIM, H_DIM), lambda i: (0, 0)),
            pl.BlockSpec((H_DIM, 1), lambda i: (0, 0)),
            pl.BlockSpec((H_DIM, 1), lambda i: (0, 0)),
        ],
        out_specs=[
            pl.BlockSpec((blk, H_DIM), lambda i: (i, 0)),
            pl.BlockSpec((blk, 1), lambda i: (i, 0)),
            pl.BlockSpec((blk, 1), lambda i: (i, 0)),
            pl.BlockSpec((8, 128), lambda i: (0, 0)),
        ],
        out_shape=[
            jax.ShapeDtypeStruct((n, H_DIM), jnp.float32),
            jax.ShapeDtypeStruct((n, 1), jnp.float32),
            jax.ShapeDtypeStruct((n, 1), jnp.float32),
            jax.ShapeDtypeStruct((8, 128), jnp.float32),
        ],
        scratch_shapes=[pltpu.SMEM((2,), jnp.float32)],
    )(x, W, a_s.reshape(H_DIM, 1), a_d.reshape(H_DIM, 1))


def _fin_proj_body(num_ref, den_ref, b_ref, w_ref, asw_ref, adw_ref,
                   h_ref, as_ref, ad_ref, m_ref, mx_ref):
    num = num_ref[0] + num_ref[1]
    den = den_ref[0] + den_ref[1] + 1e-16
    hprev = jnp.maximum(num / den + b_ref[...], 0.0)
    h = jnp.dot(hprev, w_ref[...], preferred_element_type=jnp.float32)
    h_ref[...] = h
    i = pl.program_id(0)
    _logits_tail(h, asw_ref, adw_ref, as_ref, ad_ref, m_ref, mx_ref,
                 i, pl.num_programs(0) - 1)


def _finalize_project(numer, denom, b, W, a_s, a_d):
    blk = 1024
    return pl.pallas_call(
        _fin_proj_body,
        grid=(NPAD // blk,),
        in_specs=[
            pl.BlockSpec((2, blk, H_DIM), lambda i: (0, i, 0)),
            pl.BlockSpec((2, blk, 1), lambda i: (0, i, 0)),
            pl.BlockSpec((1, H_DIM), lambda i: (0, 0)),
            pl.BlockSpec((H_DIM, H_DIM), lambda i: (0, 0)),
            pl.BlockSpec((H_DIM, 1), lambda i: (0, 0)),
            pl.BlockSpec((H_DIM, 1), lambda i: (0, 0)),
        ],
        out_specs=[
            pl.BlockSpec((blk, H_DIM), lambda i: (i, 0)),
            pl.BlockSpec((blk, 1), lambda i: (i, 0)),
            pl.BlockSpec((blk, 1), lambda i: (i, 0)),
            pl.BlockSpec((8, 128), lambda i: (0, 0)),
        ],
        out_shape=[
            jax.ShapeDtypeStruct((NPAD, H_DIM), jnp.float32),
            jax.ShapeDtypeStruct((NPAD, 1), jnp.float32),
            jax.ShapeDtypeStruct((NPAD, 1), jnp.float32),
            jax.ShapeDtypeStruct((8, 128), jnp.float32),
        ],
        scratch_shapes=[pltpu.SMEM((2,), jnp.float32)],
    )(numer, denom.reshape(2, NPAD, 1), b.reshape(1, H_DIM),
      W, a_s.reshape(H_DIM, 1), a_d.reshape(H_DIM, 1))


def _fin_lin_body(num_ref, den_ref, b_ref, w_ref, bl_ref, o_ref):
    num = num_ref[0] + num_ref[1]
    den = den_ref[0] + den_ref[1] + 1e-16
    hprev = jnp.maximum(num / den + b_ref[...], 0.0)
    o_ref[...] = jnp.dot(hprev, w_ref[...],
                         preferred_element_type=jnp.float32) + bl_ref[...]


def _finalize_linear(numer, denom, b, Wl, bl):
    blk = 2000
    return pl.pallas_call(
        _fin_lin_body,
        grid=(N_NODES // blk,),
        in_specs=[
            pl.BlockSpec((2, blk, H_DIM), lambda i: (0, i, 0)),
            pl.BlockSpec((2, blk, 1), lambda i: (0, i, 0)),
            pl.BlockSpec((1, H_DIM), lambda i: (0, 0)),
            pl.BlockSpec((H_DIM, OUT_D), lambda i: (0, 0)),
            pl.BlockSpec((1, OUT_D), lambda i: (0, 0)),
        ],
        out_specs=pl.BlockSpec((blk, OUT_D), lambda i: (i, 0)),
        out_shape=jax.ShapeDtypeStruct((N_NODES, OUT_D), jnp.float32),
    )(numer, denom.reshape(2, NPAD, 1), b.reshape(1, H_DIM),
      Wl, bl.reshape(1, OUT_D))


def _make_sc_edge(table_size):
    mesh = plsc.VectorSubcoreMesh(core_axis_name="c", subcore_axis_name="s")

    @functools.partial(
        pl.kernel,
        out_type=[
            jax.ShapeDtypeStruct((2, NPAD, H_DIM), jnp.float32),
            jax.ShapeDtypeStruct((2, NPAD), jnp.float32),
        ],
        mesh=mesh,
        compiler_params=pltpu.CompilerParams(needs_layout_passes=False),
        scratch_types=(
            [
                pltpu.VMEM((NPAD,), jnp.float32),
                pltpu.VMEM((NPAD,), jnp.float32),
                pltpu.VMEM((16,), jnp.float32),
                pltpu.VMEM((2, CHUNK, H_DIM), jnp.float32),
            ]
            + [pltpu.VMEM((CHUNK,), jnp.int32) for _ in range(4)]
            + [pltpu.VMEM((CHUNK,), jnp.int32) for _ in range(4)]
            + [pltpu.VMEM((CHUNK,), jnp.float32) for _ in range(4)]
            + [
                pltpu.VMEM_SHARED((NPAD, H_DIM), jnp.float32),
                pltpu.VMEM_SHARED((NPAD,), jnp.float32),
                pltpu.SemaphoreType.DMA,
                pltpu.SemaphoreType.DMA,
                pltpu.SemaphoreType.DMA,
                pltpu.SemaphoreType.DMA,
                pltpu.SemaphoreType.DMA,
                pltpu.SemaphoreType.DMA,
                pltpu.SemaphoreType.DMA,
            ]
        ),
    )
    def sc_edge(h_hbm, as_hbm, ad_hbm, m_hbm, src_hbm, dst_hbm,
                numer_out, denom_out,
                as_v, ad_v, m_v, rows_v, *rest):
        srcs = list(rest[0:4])
        dsts = list(rest[4:8])
        ws = list(rest[8:12])
        (numer_sh, denom_sh, sem_g0, sem_g1,
         si0, si1, si2, si3, sem_w) = rest[12:]
        sems = [sem_g0, sem_g1]
        sem_i = [si0, si1, si2, si3]
        cid = lax.axis_index("c")
        sid = lax.axis_index("s")
        zeros16 = jnp.zeros((16,), jnp.float32)

        pltpu.sync_copy(as_hbm, as_v.at[pl.ds(0, table_size)])
        pltpu.sync_copy(ad_hbm, ad_v.at[pl.ds(0, table_size)])
        pltpu.sync_copy(m_hbm.at[0, pl.ds(0, 16)], m_v)
        m_val = m_v[...]

        def zbody(i, _):
            rows_v[0, i // 8, pl.ds((i % 8) * 16, 16)] = zeros16
            return 0
        lax.fori_loop(0, CHUNK * 8, zbody, 0)
        for k in range(CHUNK // 16):
            ws[0][pl.ds(k * 16, 16)] = zeros16
        for t in range(RPT // CHUNK):
            pltpu.sync_copy(rows_v.at[0],
                            numer_sh.at[pl.ds(sid * RPT + t * CHUNK, CHUNK)])
            pltpu.sync_copy(ws[0],
                            denom_sh.at[pl.ds(sid * RPT + t * CHUNK, CHUNK)])
        plsc.subcore_barrier()

        base = (cid * 16 + sid) * EPW

        def load_idx(c, m4, sync):
            eb = base + c * CHUNK
            if sync:
                pltpu.sync_copy(src_hbm.at[pl.ds(eb, CHUNK)], srcs[m4])
                pltpu.sync_copy(dst_hbm.at[pl.ds(eb, CHUNK)], dsts[m4])
            else:
                pltpu.async_copy(src_hbm.at[pl.ds(eb, CHUNK)], srcs[m4],
                                 sem_i[m4])
                pltpu.async_copy(dst_hbm.at[pl.ds(eb, CHUNK)], dsts[m4],
                                 sem_i[m4])

        def drain_idx(m4):
            pltpu.make_async_copy(src_hbm.at[pl.ds(0, CHUNK)],
                                  srcs[m4], sem_i[m4]).wait()
            pltpu.make_async_copy(dst_hbm.at[pl.ds(0, CHUNK)],
                                  dsts[m4], sem_i[m4]).wait()

        def fire(m4, m2):
            pltpu.async_copy(h_hbm.at[srcs[m4]], rows_v.at[m2], sems[m2])
            for k in range(CHUNK // 16):
                s16 = srcs[m4][pl.ds(k * 16, 16)]
                d16 = dsts[m4][pl.ds(k * 16, 16)]
                z = (plsc.load_gather(as_v, [s16])
                     + plsc.load_gather(ad_v, [d16]))
                e = jnp.maximum(z, 0.2 * z)
                ws[m4][pl.ds(k * 16, 16)] = jnp.exp(e - m_val)

        def process(m4, m2):
            pltpu.make_async_copy(h_hbm.at[srcs[m4]],
                                  rows_v.at[m2], sems[m2]).wait()

            def mulbody(t, _):
                for u in range(4):
                    ei = 4 * t + u
                    wspl = plsc.load_gather(
                        ws[m4], [jnp.full((16,), ei, jnp.int32)])
                    for k in range(H_DIM // 16):
                        rows_v[m2, ei, pl.ds(k * 16, 16)] = (
                            rows_v[m2, ei, pl.ds(k * 16, 16)] * wspl)
                return 0
            lax.fori_loop(0, CHUNK // 4, mulbody, 0)

            pltpu.sync_copy(rows_v.at[m2], numer_sh.at[dsts[m4]], add=True)
            pltpu.async_copy(ws[m4], denom_sh.at[dsts[m4]], sem_w, add=True)

        def drain_w(m4):
            pltpu.make_async_copy(ws[m4], denom_sh.at[dsts[m4]],
                                  sem_w).wait()

        load_idx(0, 0, sync=True)
        load_idx(1, 1, sync=True)
        load_idx(2, 2, sync=False)
        fire(0, 0)
        fire(1, 1)

        process(0, 0)
        load_idx(3, 3, sync=False)
        drain_idx(2)
        fire(2, 0)

        def quad_body(k, _):
            c = 4 * k + 1
            for j in (1, 2, 3, 0):
                q = c + ((j - 1) % 4)
                process(j, j % 2)
                drain_w((j + 3) % 4)
                load_idx(q + 3, (j + 3) % 4, sync=False)
                drain_idx((j + 2) % 4)
                fire((j + 2) % 4, (j + 2) % 2)
            return 0
        lax.fori_loop(0, 30, quad_body, 0)

        process(1, 1)
        drain_w(0)
        load_idx(RPW - 1, 0, sync=False)
        drain_idx(3)
        fire(3, 1)
        process(2, 0)
        drain_w(1)
        drain_idx(0)
        fire(0, 0)
        process(3, 1)
        drain_w(2)
        process(0, 0)
        drain_w(3)
        drain_w(0)

        plsc.subcore_barrier()
        r0 = sid * RPT
        pltpu.sync_copy(numer_sh.at[pl.ds(r0, RPT)],
                        numer_out.at[cid, pl.ds(r0, RPT)])
        pltpu.sync_copy(denom_sh.at[pl.ds(r0, RPT)],
                        denom_out.at[cid, pl.ds(r0, RPT)])

    return sc_edge


_sc_edge_l1 = _make_sc_edge(N_NODES)
_sc_edge_l2 = _make_sc_edge(NPAD)


@jax.jit
def kernel(x, edge_index, edge_attr, batch, W1, a_s1, a_d1, b1,
           W2, a_s2, a_d2, b2, Wl, bl):
    src = edge_index[0]
    dst = edge_index[1]

    h1, as1, ad1, m1 = _project(x, W1, a_s1, a_d1)
    numer1, denom1 = _sc_edge_l1(h1, as1.reshape(-1), ad1.reshape(-1), m1,
                                 src, dst)
    h2, as2, ad2, m2 = _finalize_project(numer1, denom1, b1, W2, a_s2, a_d2)
    numer2, denom2 = _sc_edge_l2(h2, as2.reshape(-1), ad2.reshape(-1), m2,
                                 src, dst)
    out = _finalize_linear(numer2, denom2, b2, Wl, bl)
    return out.reshape(-1).reshape(-1, GENES_X_DIM)
